# speedup vs baseline: 5.0003x; 5.0003x over previous
import functools

import jax
import jax.numpy as jnp
from jax import lax
from jax.experimental import pallas as pl
from jax.experimental.pallas import tpu as pltpu
from jax.experimental.pallas import tpu_sc as plsc

NC, NS, L = 2, 16, 16
NW = NC * NS
NE, D = 32, 128
NV = 64
B, C = 1024, 200
N = B * C
TPW = N // NW
K = 128
NCHUNK = TPW // K

_MULT = [0.5 + 0.0625 * (i % 16) for i in range(NV)]

_mesh = plsc.VectorSubcoreMesh(
    core_axis_name="c", subcore_axis_name="s", num_cores=NC, num_subcores=NS
)


@functools.partial(
    pl.kernel,
    out_type=jax.ShapeDtypeStruct((N, D), jnp.float32),
    mesh=_mesh,
    scratch_types=[
        pltpu.VMEM_SHARED((NV, D), jnp.float32),
        pltpu.VMEM((D,), jnp.float32),
        pltpu.VMEM((K,), jnp.int32),
        pltpu.VMEM((K, D), jnp.float32),
        pltpu.SemaphoreType.DMA,
    ],
)
def _vme_kernel(in_hbm, emb_hbm, out_hbm, table_sh, row_v, idx_v, rows_v, sem):
    cid = lax.axis_index("c")
    sid = lax.axis_index("s")
    wid = sid * NC + cid

    @pl.when(sid == 0)
    def _build():
        for r in range(NV):
            pltpu.sync_copy(emb_hbm.at[r % NE], row_v)
            for j in range(D // L):
                sl = pl.ds(j * L, L)
                row_v[sl] = row_v[sl] * _MULT[r]
            pltpu.sync_copy(row_v, table_sh.at[r])

    plsc.subcore_barrier()

    base = wid * TPW

    def chunk(i, carry):
        off = base + i * K
        pltpu.sync_copy(in_hbm.at[pl.ds(off, K)], idx_v)
        pltpu.async_copy(table_sh.at[idx_v], rows_v, sem).wait()
        pltpu.sync_copy(rows_v, out_hbm.at[pl.ds(off, K)])
        return carry

    lax.fori_loop(0, NCHUNK, chunk, 0)


def kernel(input_BC, raw_embed):
    out = _vme_kernel(input_BC.reshape(N), raw_embed)
    return out.reshape(B, C, D)


# speedup vs baseline: 7.8825x; 1.5764x over previous
import functools

import jax
import jax.numpy as jnp
from jax import lax
from jax.experimental import pallas as pl
from jax.experimental.pallas import tpu as pltpu
from jax.experimental.pallas import tpu_sc as plsc

NC, NS, L = 2, 16, 16
NW = NC * NS
NE, D = 32, 128
NV = 64
B, C = 1024, 200
N = B * C
TPW = N // NW
K = 80
NCHUNK = TPW // K
NB = 4
NSTEP = NCHUNK // NB

_MULT = [0.5 + 0.0625 * (i % 16) for i in range(NV)]

_mesh = plsc.VectorSubcoreMesh(
    core_axis_name="c", subcore_axis_name="s", num_cores=NC, num_subcores=NS
)


@functools.partial(
    pl.kernel,
    out_type=jax.ShapeDtypeStruct((N, D), jnp.float32),
    mesh=_mesh,
    scratch_types=[
        pltpu.VMEM_SHARED((NV, D), jnp.float32),
        pltpu.VMEM((D,), jnp.float32),
        pltpu.VMEM((TPW,), jnp.int32),
        [pltpu.VMEM((K, D), jnp.float32)] * NB,
        [pltpu.SemaphoreType.DMA] * NB,
        [pltpu.SemaphoreType.DMA] * NB,
    ],
)
def _vme_kernel(
    in_hbm, emb_hbm, out_hbm, table_sh, row_v, idx_all, rows_v, gsem, osem
):
    cid = lax.axis_index("c")
    sid = lax.axis_index("s")
    wid = sid * NC + cid

    @pl.when(sid == 0)
    def _build():
        for r in range(NV):
            pltpu.sync_copy(emb_hbm.at[r % NE], row_v)
            for j in range(D // L):
                sl = pl.ds(j * L, L)
                row_v[sl] = row_v[sl] * _MULT[r]
            pltpu.sync_copy(row_v, table_sh.at[r])

    plsc.subcore_barrier()

    base = wid * TPW
    pltpu.sync_copy(in_hbm.at[pl.ds(base, TPW)], idx_all)

    def g_copy(j, b):
        return pltpu.make_async_copy(
            table_sh.at[idx_all.at[pl.ds(j * K, K)]], rows_v[b], gsem[b]
        )

    def w_copy(j, b):
        return pltpu.make_async_copy(
            rows_v[b], out_hbm.at[pl.ds(base + j * K, K)], osem[b]
        )

    for b in range(NB):
        g_copy(b, b).start()

    def step(s, carry):
        for b in range(NB):
            j = s * NB + b
            g_copy(j, b).wait()
            w_copy(j, b).start()
            w_copy(j, b).wait()
            jn = j + NB

            @pl.when(jn < NCHUNK)
            def _next():
                g_copy(jn, b).start()

        return carry

    lax.fori_loop(0, NSTEP, step, 0)


def kernel(input_BC, raw_embed):
    out = _vme_kernel(input_BC.reshape(N), raw_embed)
    return out.reshape(B, C, D)
